# baseline (speedup 1.0000x reference)
import functools

import jax
import jax.numpy as jnp
from jax import lax
from jax.experimental import pallas as pl
from jax.experimental.pallas import tpu as pltpu
from jax.experimental.pallas import tpu_sc as plsc

_NW = 32
_LANES = 128
_SUB = 8
_ILEN = 56


@functools.partial(jax.jit, static_argnums=(2, 3, 4))
def _gather_rows(idx8, seg_tab, n_chunks, chunk, d):
    n_main = d // _LANES
    d_main = n_main * _LANES
    d_tail = d - d_main
    mesh = plsc.VectorSubcoreMesh(core_axis_name="c", subcore_axis_name="s")

    @functools.partial(
        pl.kernel,
        mesh=mesh,
        out_type=jax.ShapeDtypeStruct((_NW * n_chunks, chunk, d), jnp.float32),
        scratch_types=[
            pltpu.VMEM((chunk, d), jnp.float32),
            pltpu.VMEM((chunk, d), jnp.float32),
            pltpu.VMEM((_ILEN, _LANES), jnp.float32),
            pltpu.VMEM((_ILEN, _LANES), jnp.float32),
            pltpu.VMEM((8 * _ILEN,), jnp.int32),
            pltpu.VMEM((8 * _ILEN,), jnp.int32),
            pltpu.SemaphoreType.DMA,
            pltpu.SemaphoreType.DMA,
            pltpu.SemaphoreType.DMA,
            pltpu.SemaphoreType.DMA,
            pltpu.SemaphoreType.DMA,
            pltpu.SemaphoreType.DMA,
            pltpu.SemaphoreType.DMA,
            pltpu.SemaphoreType.DMA,
        ],
        compiler_params=pltpu.CompilerParams(needs_layout_passes=False),
    )
    def gather_kernel(
        idx_hbm, tab_hbm, out_hbm,
        rows0, rows1, tail0, tail1, idx0, idx1,
        g0, g1, t0, t1, s0, s1, i0, i1,
    ):
        cid = lax.axis_index("c")
        sid = lax.axis_index("s")
        wid = sid * 2 + cid
        base = wid * n_chunks
        rows = (rows0, rows1)
        tails = (tail0, tail1)
        idxc = (idx0, idx1)
        gsem = (g0, g1)
        tsem = (t0, t1)
        ssem = (s0, s1)
        isem = (i0, i1)

        def idx_slice(j):
            return idx_hbm.at[pl.ds((base + j) * (8 * _ILEN), 8 * _ILEN)]

        def main_gathers(j, b, op):
            for jj in range(n_main):
                op(
                    tab_hbm.at[idxc[b].at[pl.ds(jj * _ILEN, chunk)]],
                    rows[b].at[:, pl.ds(jj * _LANES, _LANES)],
                    gsem[b],
                )

        def tail_gather(b, op):
            op(tab_hbm.at[idxc[b].at[pl.ds(n_main * _ILEN, _ILEN)]], tails[b], tsem[b])

        def scatter(j, b, op):
            op(rows[b], out_hbm.at[base + j], ssem[b])

        def start(src, dst, sem):
            pltpu.async_copy(src, dst, sem)

        def wait(src, dst, sem):
            pltpu.make_async_copy(src, dst, sem).wait()

        def tail_copy(b):
            lane = lax.iota(jnp.int32, 16)
            n_grp = (d_tail + 15) // 16

            def body(r, carry):
                r_vec = jnp.full((16,), r, dtype=jnp.int32)
                for k in range(n_grp):
                    msk = 16 * k + lane < d_tail
                    vals = plsc.load_gather(
                        tails[b], [r_vec, 16 * k + lane], mask=msk
                    )
                    plsc.store_scatter(
                        rows[b], [r_vec, d_main + 16 * k + lane], vals, mask=msk
                    )
                return carry

            lax.fori_loop(0, chunk, body, 0)

        pltpu.sync_copy(idx_slice(0), idxc[0])
        start(idx_slice(1), idxc[1], isem[1])
        main_gathers(0, 0, start)
        tail_gather(0, start)

        def body(j, carry):
            b = lax.rem(j, 2)
            for bb in (0, 1):
                @pl.when(b == bb)
                def _():
                    _step(j, bb)
            return carry

        def _step(j, b):
            o = 1 - b
            @pl.when(j > 0)
            def _():
                scatter(j - 1, o, wait)

            main_gathers(j, b, wait)
            tail_gather(b, wait)
            tail_copy(b)
            @pl.when(j + 1 < n_chunks)
            def _():
                wait(idx_slice(j + 1), idxc[o], isem[o])
                main_gathers(j + 1, o, start)
                tail_gather(o, start)

            scatter(j, b, start)
            @pl.when(j + 2 < n_chunks)
            def _():
                start(idx_slice(j + 2), idxc[b], isem[b])

        lax.fori_loop(0, n_chunks, body, 0)
        scatter(n_chunks - 1, (n_chunks - 1) % 2, wait)

    return gather_kernel(idx8, seg_tab)


def kernel(x, embedding_table):
    b, s = x.shape
    v, d = embedding_table.shape
    chunk = s
    n_chunks = b // _NW
    d_pad = ((d + _LANES - 1) // _LANES) * _LANES
    seg_tab = (
        jnp.pad(embedding_table, ((0, 0), (0, d_pad - d)))
        .reshape(v * (d_pad // _LANES), _LANES)
    )
    xi = x.astype(jnp.int32)
    idx8 = 8 * xi[:, None, :] + jnp.arange(8, dtype=jnp.int32)[None, :, None]
    idx8 = jnp.pad(idx8, ((0, 0), (0, 0), (0, _ILEN - s))).reshape(-1)
    return _gather_rows(idx8, seg_tab, n_chunks, chunk, d)


# speedup vs baseline: 1.9256x; 1.9256x over previous
import functools

import jax
import jax.numpy as jnp
from jax import lax
from jax.experimental import pallas as pl
from jax.experimental.pallas import tpu as pltpu
from jax.experimental.pallas import tpu_sc as plsc

_NW = 32
_LANES = 128
_SUB = 8
_ILEN = 56


@functools.partial(jax.jit, static_argnums=(4, 5, 6))
def _gather_rows(idx1, tab_main, tab_tail, seg_tab, n_chunks, chunk, d):
    n_main = d // _LANES
    d_main = n_main * _LANES
    d_tail = d - d_main
    c_main = (chunk // _SUB) * _SUB
    n_fix = (chunk - c_main) * 8
    mesh = plsc.VectorSubcoreMesh(core_axis_name="c", subcore_axis_name="s")

    @functools.partial(
        pl.kernel,
        mesh=mesh,
        out_type=jax.ShapeDtypeStruct((_NW * n_chunks, chunk, d), jnp.float32),
        scratch_types=[
            pltpu.VMEM((chunk, d), jnp.float32),
            pltpu.VMEM((chunk, d), jnp.float32),
            pltpu.VMEM((c_main, _LANES), jnp.float32),
            pltpu.VMEM((n_fix, _LANES), jnp.float32),
            pltpu.VMEM((80,), jnp.int32),
            pltpu.VMEM((80,), jnp.int32),
            pltpu.SemaphoreType.DMA,
            pltpu.SemaphoreType.DMA,
            pltpu.SemaphoreType.DMA,
            pltpu.SemaphoreType.DMA,
            pltpu.SemaphoreType.DMA,
            pltpu.SemaphoreType.DMA,
            pltpu.SemaphoreType.DMA,
            pltpu.SemaphoreType.DMA,
        ],
        compiler_params=pltpu.CompilerParams(needs_layout_passes=False),
    )
    def gather_kernel(
        idx_hbm, tabm_hbm, tabt_hbm, segt_hbm, out_hbm,
        rows0, rows1, tail_v, fix_v, idx0, idx1,
        g0, g1, tsem, fsem, s0, s1, i0, i1,
    ):
        cid = lax.axis_index("c")
        sid = lax.axis_index("s")
        wid = sid * 2 + cid
        base = wid * n_chunks
        rows = (rows0, rows1)
        idxc = (idx0, idx1)
        gsem = (g0, g1)
        ssem = (s0, s1)
        isem = (i0, i1)

        def idx_slice(j):
            return idx_hbm.at[pl.ds((base + j) * 80, 80)]

        def main_gathers(j, b, op):
            op(
                tabm_hbm.at[idxc[b].at[pl.ds(0, c_main)]],
                rows[b].at[pl.ds(0, c_main), pl.ds(0, d_main)],
                gsem[b],
            )

        def tail_gather(b, op):
            op(tabt_hbm.at[idxc[b].at[pl.ds(0, c_main)]], tail_v, tsem)
            op(segt_hbm.at[idxc[b].at[pl.ds(_ILEN, n_fix)]], fix_v, fsem)

        def scatter(j, b, op):
            op(rows[b], out_hbm.at[base + j], ssem[b])

        def start(src, dst, sem):
            pltpu.async_copy(src, dst, sem)

        def wait(src, dst, sem):
            pltpu.make_async_copy(src, dst, sem).wait()

        def tail_copy(b):
            lane = lax.iota(jnp.int32, 16)
            n_grp = (d_tail + 15) // 16

            def body(r, carry):
                r_vec = jnp.full((16,), r, dtype=jnp.int32)
                for k in range(n_grp):
                    msk = 16 * k + lane < d_tail
                    vals = plsc.load_gather(tail_v, [r_vec, 16 * k + lane], mask=msk)
                    plsc.store_scatter(
                        rows[b], [r_vec, d_main + 16 * k + lane], vals, mask=msk
                    )
                return carry

            lax.fori_loop(0, c_main, body, 0)

            for r in range(chunk - c_main):
                for j in range(8):
                    e_vec = jnp.full((16,), r * 8 + j, dtype=jnp.int32)
                    r_vec = jnp.full((16,), c_main + r, dtype=jnp.int32)
                    for g in range(8):
                        col = j * _LANES + 16 * g
                        if col >= d:
                            continue
                        msk = col + lane < d
                        vals = plsc.load_gather(fix_v, [e_vec, 16 * g + lane], mask=msk)
                        plsc.store_scatter(
                            rows[b], [r_vec, col + lane], vals, mask=msk
                        )

        pltpu.sync_copy(idx_slice(0), idxc[0])
        start(idx_slice(1), idxc[1], isem[1])
        main_gathers(0, 0, start)
        tail_gather(0, start)

        def body(j, carry):
            b = lax.rem(j, 2)
            for bb in (0, 1):
                @pl.when(b == bb)
                def _():
                    _step(j, bb)
            return carry

        def _step(j, b):
            o = 1 - b
            @pl.when(j > 0)
            def _():
                scatter(j - 1, o, wait)

            main_gathers(j, b, wait)
            tail_gather(b, wait)
            tail_copy(b)
            @pl.when(j + 1 < n_chunks)
            def _():
                wait(idx_slice(j + 1), idxc[o], isem[o])
                main_gathers(j + 1, o, start)
                tail_gather(o, start)

            scatter(j, b, start)
            @pl.when(j + 2 < n_chunks)
            def _():
                start(idx_slice(j + 2), idxc[b], isem[b])

        lax.fori_loop(0, n_chunks, body, 0)
        scatter(n_chunks - 1, (n_chunks - 1) % 2, wait)

    return gather_kernel(idx1, tab_main, tab_tail, seg_tab)


def kernel(x, embedding_table):
    b, s = x.shape
    v, d = embedding_table.shape
    chunk = s
    n_chunks = b // _NW
    d_main = (d // _LANES) * _LANES
    d_pad = ((d + _LANES - 1) // _LANES) * _LANES
    c_main = (s // _SUB) * _SUB
    tab_main = embedding_table[:, :d_main]
    tab_tail = jnp.pad(embedding_table[:, d_main:], ((0, 0), (0, d_pad - d)))
    seg_tab = (
        jnp.pad(embedding_table, ((0, 0), (0, d_pad - d)))
        .reshape(v * (d_pad // _LANES), _LANES)
    )
    xi = x.astype(jnp.int32)
    nseg = d_pad // _LANES
    fix = nseg * xi[:, c_main:, None] + jnp.arange(nseg, dtype=jnp.int32)[None, None, :]
    fix = fix.reshape(b, (s - c_main) * nseg)
    blk = jnp.concatenate(
        [
            jnp.pad(xi, ((0, 0), (0, _ILEN - s))),
            fix,
            jnp.zeros((b, 80 - _ILEN - (s - c_main) * nseg), jnp.int32),
        ],
        axis=1,
    )
    idx1 = blk.reshape(-1)
    return _gather_rows(idx1, tab_main, tab_tail, seg_tab, n_chunks, chunk, d)
